# baseline (speedup 1.0000x reference)
import jax
import jax.numpy as jnp
from jax.experimental import pallas as pl
from jax.experimental.pallas import tpu as pltpu

_NBUF = 8
_C = 4


def _moe_kernel(idx_ref, x_ref, w_hbm, b_ref, out_ref, w_buf, sem):
    E, D, O = w_hbm.shape
    DC = D // _C
    TOT = E * _C

    def make_copy(t, slot):
        e = t // _C
        c = jax.lax.rem(t, _C)
        return pltpu.make_async_copy(
            w_hbm.at[e, pl.ds(c * DC, DC), :],
            w_buf.at[slot],
            sem.at[slot],
        )

    for s in range(_NBUF):
        make_copy(s, s).start()

    def body(r, _):
        for s in range(_NBUF):
            t = r * _NBUF + s
            e = t // _C
            c = jax.lax.rem(t, _C)
            make_copy(t, s).wait()
            mask = (idx_ref[...] == e).astype(jnp.float32)
            xm = x_ref[:, pl.ds(c * DC, DC)] * mask
            part = jnp.dot(xm, w_buf[s], preferred_element_type=jnp.float32)
            part = jnp.where(c == _C - 1, part + mask * b_ref[e], part)

            @pl.when(t == 0)
            def _init():
                out_ref[...] = part

            @pl.when(t != 0)
            def _accum():
                out_ref[...] += part

            nxt = t + _NBUF

            @pl.when(nxt < TOT)
            def _prefetch():
                make_copy(nxt, s).start()

        return 0

    jax.lax.fori_loop(0, TOT // _NBUF, body, 0)


def kernel(inputs, indices, w, b):
    N, D = inputs.shape
    E, _, O = w.shape
    idx2d = indices.astype(jnp.int32).reshape(N, 1)
    b3d = b.reshape(E, 1, O)

    return pl.pallas_call(
        _moe_kernel,
        in_specs=[
            pl.BlockSpec(memory_space=pltpu.VMEM),
            pl.BlockSpec(memory_space=pltpu.VMEM),
            pl.BlockSpec(memory_space=pl.ANY),
            pl.BlockSpec(memory_space=pltpu.VMEM),
        ],
        out_specs=pl.BlockSpec(memory_space=pltpu.VMEM),
        out_shape=jax.ShapeDtypeStruct((N, O), jnp.float32),
        scratch_shapes=[
            pltpu.VMEM((_NBUF, D // _C, O), jnp.float32),
            pltpu.SemaphoreType.DMA((_NBUF,)),
        ],
    )(idx2d, inputs, w, b3d)


# speedup vs baseline: 1.0510x; 1.0510x over previous
import jax
import jax.numpy as jnp
from jax.experimental import pallas as pl
from jax.experimental.pallas import tpu as pltpu

_NBUF = 4
_C = 1


def _moe_kernel(idx_ref, x_ref, w_hbm, b_ref, out_ref, w_buf, sem):
    E, D, O = w_hbm.shape
    DC = D // _C
    TOT = E * _C

    def make_copy(t, slot):
        e = t // _C
        c = jax.lax.rem(t, _C)
        return pltpu.make_async_copy(
            w_hbm.at[e, pl.ds(c * DC, DC), :],
            w_buf.at[slot],
            sem.at[slot],
        )

    for s in range(_NBUF):
        make_copy(s, s).start()

    def body(r, _):
        for s in range(_NBUF):
            t = r * _NBUF + s
            e = t // _C
            c = jax.lax.rem(t, _C)
            make_copy(t, s).wait()
            mask = (idx_ref[...] == e).astype(jnp.float32)
            xm = x_ref[:, pl.ds(c * DC, DC)] * mask
            part = jnp.dot(xm, w_buf[s], preferred_element_type=jnp.float32)
            part = jnp.where(c == _C - 1, part + mask * b_ref[e], part)

            @pl.when(t == 0)
            def _init():
                out_ref[...] = part

            @pl.when(t != 0)
            def _accum():
                out_ref[...] += part

            nxt = t + _NBUF

            @pl.when(nxt < TOT)
            def _prefetch():
                make_copy(nxt, s).start()

        return 0

    jax.lax.fori_loop(0, TOT // _NBUF, body, 0)


def kernel(inputs, indices, w, b):
    N, D = inputs.shape
    E, _, O = w.shape
    idx2d = indices.astype(jnp.int32).reshape(N, 1)
    b3d = b.reshape(E, 1, O)

    return pl.pallas_call(
        _moe_kernel,
        in_specs=[
            pl.BlockSpec(memory_space=pltpu.VMEM),
            pl.BlockSpec(memory_space=pltpu.VMEM),
            pl.BlockSpec(memory_space=pl.ANY),
            pl.BlockSpec(memory_space=pltpu.VMEM),
        ],
        out_specs=pl.BlockSpec(memory_space=pltpu.VMEM),
        out_shape=jax.ShapeDtypeStruct((N, O), jnp.float32),
        scratch_shapes=[
            pltpu.VMEM((_NBUF, D // _C, O), jnp.float32),
            pltpu.SemaphoreType.DMA((_NBUF,)),
        ],
    )(idx2d, inputs, w, b3d)


# speedup vs baseline: 1.0544x; 1.0032x over previous
import jax
import jax.numpy as jnp
from jax.experimental import pallas as pl
from jax.experimental.pallas import tpu as pltpu

_NBUF = 3
_SUB = 4


def _moe_kernel(idx_ref, x_ref, w_hbm, b_ref, out_ref, w_buf, sem):
    E, D, O = w_hbm.shape
    DS = D // _SUB

    def copies(e, slot):
        return [
            pltpu.make_async_copy(
                w_hbm.at[e, pl.ds(u * DS, DS), :],
                w_buf.at[slot, pl.ds(u * DS, DS), :],
                sem.at[slot, u],
            )
            for u in range(_SUB)
        ]

    for s in range(_NBUF):
        for cp in copies(s, s):
            cp.start()

    def step(e, s):
        for cp in copies(e, s):
            cp.wait()
        mask = (idx_ref[...] == e).astype(jnp.float32)
        xm = x_ref[...] * mask
        part = jnp.dot(xm, w_buf[s], preferred_element_type=jnp.float32)
        part = part + mask * b_ref[e]

        @pl.when(e == 0)
        def _init():
            out_ref[...] = part

        @pl.when(e != 0)
        def _accum():
            out_ref[...] += part

        nxt = e + _NBUF

        @pl.when(nxt < E)
        def _prefetch():
            for cp in copies(nxt, s):
                cp.start()

    def body(r, _):
        for s in range(_NBUF):
            step(r * _NBUF + s, s)
        return 0

    rounds = E // _NBUF
    jax.lax.fori_loop(0, rounds, body, 0)
    for s in range(E - rounds * _NBUF):
        step(rounds * _NBUF + s, s)


def kernel(inputs, indices, w, b):
    N, D = inputs.shape
    E, _, O = w.shape
    idx2d = indices.astype(jnp.int32).reshape(N, 1)
    b3d = b.reshape(E, 1, O)

    return pl.pallas_call(
        _moe_kernel,
        in_specs=[
            pl.BlockSpec(memory_space=pltpu.VMEM),
            pl.BlockSpec(memory_space=pltpu.VMEM),
            pl.BlockSpec(memory_space=pl.ANY),
            pl.BlockSpec(memory_space=pltpu.VMEM),
        ],
        out_specs=pl.BlockSpec(memory_space=pltpu.VMEM),
        out_shape=jax.ShapeDtypeStruct((N, O), jnp.float32),
        scratch_shapes=[
            pltpu.VMEM((_NBUF, D, O), jnp.float32),
            pltpu.SemaphoreType.DMA((_NBUF, _SUB)),
        ],
    )(idx2d, inputs, w, b3d)
